# baseline (speedup 1.0000x reference)
import functools

import jax
import jax.numpy as jnp
from jax import lax
from jax.experimental import pallas as pl
from jax.experimental.pallas import tpu as pltpu
from jax.experimental.pallas import tpu_sc as plsc

N = 10000
E = 320000
F = 128
NPAD = 10112
DUMMY = 10008
RPT = NPAD // 16
BW = 128
NB = 160
CH = 32
NCH = NB // CH
PPT = NB * BW
NB_SLOW = 64
NB_FAST = 2 * NB - NB_SLOW
SLOW_CORE = 0
IDXROWS = 2 * 16 * NB
NTILES = 32
BL = 400
GS = 40
PAIRS = 125


def _spmm_sc(a, src_idx, dst_idx, zeros):
    mesh = plsc.VectorSubcoreMesh(core_axis_name="c", subcore_axis_name="s")

    @functools.partial(
        pl.kernel,
        mesh=mesh,
        out_type=jax.ShapeDtypeStruct((2, NPAD, F), jnp.float32),
        scratch_types=[
            pltpu.VMEM((CH, BW), jnp.int32),
            pltpu.VMEM((CH, BW), jnp.int32),
            pltpu.VMEM((BW, F), jnp.float32),
            pltpu.VMEM((BW, F), jnp.float32),
            pltpu.VMEM_SHARED((NPAD, F), jnp.float32),
            pltpu.SemaphoreType.DMA,
            pltpu.SemaphoreType.DMA,
            pltpu.SemaphoreType.DMA,
            pltpu.SemaphoreType.DMA,
        ],
    )
    def k(a_hbm, src_hbm, dst_hbm, z_hbm, out_hbm, src_v, dst_v, rows0, rows1,
          acc, gsem0, gsem1, ssem0, ssem1):
        c = lax.axis_index("c")
        s = lax.axis_index("s")
        r0 = s * RPT
        pltpu.sync_copy(z_hbm.at[pl.ds(r0, RPT)], acc.at[pl.ds(r0, RPT)])
        plsc.subcore_barrier()
        on_slow = c == SLOW_CORE
        base = jnp.where(on_slow, s * NB_SLOW, 16 * NB_SLOW + s * NB_FAST)
        nch = jnp.where(on_slow, NB_SLOW // CH, NB_FAST // CH)

        def chunk(cc, carry):
            pltpu.sync_copy(src_hbm.at[pl.ds(base + cc * CH, CH)], src_v)
            pltpu.sync_copy(dst_hbm.at[pl.ds(base + cc * CH, CH)], dst_v)
            pltpu.async_copy(a_hbm.at[src_v.at[0]], rows0, gsem0)
            pltpu.async_copy(a_hbm.at[src_v.at[1]], rows1, gsem1)

            def m_body(m, carry2):
                j0 = 2 * m
                pltpu.make_async_copy(a_hbm.at[src_v.at[j0]], rows0, gsem0).wait()
                pltpu.async_copy(rows0, acc.at[dst_v.at[j0]], ssem0, add=True)
                pltpu.make_async_copy(a_hbm.at[src_v.at[j0 + 1]], rows1, gsem1).wait()
                pltpu.async_copy(rows1, acc.at[dst_v.at[j0 + 1]], ssem1, add=True)

                @pl.when(m < CH // 2 - 1)
                def _():
                    pltpu.make_async_copy(rows0, acc.at[dst_v.at[j0]], ssem0).wait()
                    pltpu.async_copy(a_hbm.at[src_v.at[j0 + 2]], rows0, gsem0)
                    pltpu.make_async_copy(rows1, acc.at[dst_v.at[j0 + 1]], ssem1).wait()
                    pltpu.async_copy(a_hbm.at[src_v.at[j0 + 3]], rows1, gsem1)

                return carry2

            lax.fori_loop(0, CH // 2, m_body, 0)
            pltpu.make_async_copy(rows0, acc.at[dst_v.at[0]], ssem0).wait()
            pltpu.make_async_copy(rows1, acc.at[dst_v.at[1]], ssem1).wait()
            return carry

        lax.fori_loop(0, nch, chunk, 0)
        plsc.subcore_barrier()
        pltpu.sync_copy(acc.at[pl.ds(r0, RPT)], out_hbm.at[c, pl.ds(r0, RPT)])

    return k(a, src_idx, dst_idx, zeros)


def _node_spec():
    return pl.BlockSpec((BL, F), lambda i: (i, 0))


def _full_spec(shape):
    return pl.BlockSpec(shape, lambda i: tuple(0 for _ in shape))


def _tc_pre(nf, efa0, efa1, wen, ben, wee, bee, wmsg, bmsg):
    f32 = jnp.float32

    def body(nf_r, e0_r, e1_r, wen_r, ben_r, wee_r, bee_r, wmsg_r, bmsg_r,
             h_o, a_o, b_o, kc_o, dg_o):
        h0 = jnp.dot(nf_r[:], wen_r[:], preferred_element_type=f32) + ben_r[:]
        w1 = wmsg_r[0:128, :]
        w2 = wmsg_r[128:256, :]
        w3 = wmsg_r[256:272, :]
        ea = e0_r[:] + e1_r[:]
        we3 = jnp.dot(wee_r[:], w3, preferred_element_type=f32)
        bb = jnp.dot(bee_r[:], w3, preferred_element_type=f32) + bmsg_r[:]
        m_kc = jnp.concatenate([we3, bb, jnp.zeros((111, F), f32)], axis=0)
        m_dg = jnp.concatenate(
            [jnp.zeros((16, F), f32), jnp.ones((1, F), f32), jnp.zeros((111, F), f32)],
            axis=0)
        kc_o[:] = jnp.dot(ea, m_kc, preferred_element_type=f32)
        dg_o[:] = jnp.dot(ea, m_dg, preferred_element_type=f32)
        h_o[:] = h0
        a_o[:] = jnp.dot(h0, w1, preferred_element_type=f32)
        b_o[:] = jnp.dot(h0, w2, preferred_element_type=f32)

    out = jax.ShapeDtypeStruct((N, F), f32)
    return pl.pallas_call(
        body,
        grid=(N // BL,),
        in_specs=[
            _node_spec(),
            _node_spec(),
            _node_spec(),
            _full_spec((F, F)),
            _full_spec((1, F)),
            _full_spec((16, 16)),
            _full_spec((1, 16)),
            _full_spec((272, F)),
            _full_spec((1, F)),
        ],
        out_specs=[_node_spec()] * 5,
        out_shape=[out] * 5,
    )(nf, efa0, efa1, wen, ben, wee, bee, wmsg, bmsg)


def _tc_update(s0, s1, b, h, kc, dg, wupd, bupd, wmsg):
    f32 = jnp.float32

    def body(s0_r, s1_r, b_r, h_r, kc_r, dg_r, wupd_r, bupd_r, wmsg_r,
             h_o, a_o, b_o):
        agg = s0_r[:] + s1_r[:] + kc_r[:] + dg_r[:] * b_r[:]
        hn = (jnp.dot(agg, wupd_r[0:128, :], preferred_element_type=f32)
              + jnp.dot(h_r[:], wupd_r[128:256, :], preferred_element_type=f32)
              + bupd_r[:])
        h_o[:] = hn
        a_o[:] = jnp.dot(hn, wmsg_r[0:128, :], preferred_element_type=f32)
        b_o[:] = jnp.dot(hn, wmsg_r[128:256, :], preferred_element_type=f32)

    out = jax.ShapeDtypeStruct((N, F), f32)
    return pl.pallas_call(
        body,
        grid=(N // BL,),
        in_specs=[
            _node_spec(), _node_spec(), _node_spec(), _node_spec(),
            _node_spec(), _node_spec(),
            _full_spec((256, F)),
            _full_spec((1, F)),
            _full_spec((272, F)),
        ],
        out_specs=[_node_spec()] * 3,
        out_shape=[out] * 3,
    )(s0, s1, b, h, kc, dg, wupd, bupd, wmsg)


def _tc_final(h, wagg, bagg):
    f32 = jnp.float32

    def body(h_r, wagg_r, bagg_r, o_r):
        g = jnp.dot(h_r[:], wagg_r[:], preferred_element_type=f32) + bagg_r[:]
        gates = jax.nn.sigmoid(g[:, 0:F])
        feat = g[:, F:2 * F] * gates
        q = feat[0:GS, :]
        c = feat[GS:2 * GS, :]
        sc = lax.dot_general(q, c, (((1,), (1,)), ((), ())),
                             preferred_element_type=f32)
        tot = jnp.sum(jnp.max(sc, axis=1))
        o_r[:] = jnp.full((1, 8, F), tot, f32)

    return pl.pallas_call(
        body,
        grid=(PAIRS,),
        in_specs=[
            pl.BlockSpec((2 * GS, F), lambda i: (i, 0)),
            _full_spec((F, 2 * F)),
            _full_spec((1, 2 * F)),
        ],
        out_specs=pl.BlockSpec((1, 8, F), lambda i: (i, 0, 0)),
        out_shape=jax.ShapeDtypeStruct((PAIRS, 8, F), f32),
    )(h, wagg, bagg)


def kernel(node_features, edge_features, from_idx, to_idx, graph_idx,
           batch_data_sizes, W_enc_node, b_enc_node, W_enc_edge, b_enc_edge,
           W_msg, b_msg, W_upd, b_upd, W_agg, b_agg):
    f32 = jnp.float32
    i32 = jnp.int32
    ben = b_enc_node.reshape(1, F)
    bee = b_enc_edge.reshape(1, 16)
    bmsg = b_msg.reshape(1, F)
    bupd = b_upd.reshape(1, F)
    bagg = b_agg.reshape(1, 2 * F)

    npair_pad = NTILES * PPT - 2 * E
    src_all = jnp.concatenate(
        [from_idx, to_idx, jnp.zeros((npair_pad,), i32)]).reshape(IDXROWS, BW)
    dst_all = jnp.concatenate(
        [to_idx, from_idx, jnp.full((npair_pad,), DUMMY, i32)]).reshape(IDXROWS, BW)

    eids = jnp.arange(E, dtype=i32)
    esrc = jnp.concatenate(
        [eids, eids, jnp.zeros((npair_pad,), i32)]).reshape(IDXROWS, BW)
    ef128 = jnp.concatenate(
        [edge_features, jnp.ones((E, 1), f32), jnp.zeros((E, F - 17), f32)], axis=1)

    z_f = jnp.zeros((NPAD, F), f32)

    efa = _spmm_sc(ef128, esrc, dst_all, z_f)
    h, a, b, kc, dg = _tc_pre(node_features, efa[0], efa[1], W_enc_node, ben,
                              W_enc_edge, bee, W_msg, bmsg)
    for _ in range(3):
        s = _spmm_sc(a, src_all, dst_all, z_f)
        h, a, b = _tc_update(s[0], s[1], b, h, kc, dg, W_upd, bupd, W_msg)
    out = _tc_final(h, W_agg, bagg)
    return out[:, 0, 0]


# speedup vs baseline: 1.0249x; 1.0249x over previous
import functools

import jax
import jax.numpy as jnp
from jax import lax
from jax.experimental import pallas as pl
from jax.experimental.pallas import tpu as pltpu
from jax.experimental.pallas import tpu_sc as plsc

N = 10000
E = 320000
F = 128
NPAD = 10112
DUMMY = 10008
RPT = NPAD // 16
BW = 128
NB = 160
CH = 32
NCH = NB // CH
PPT = NB * BW
NB_SLOW = 64
NB_FAST = 2 * NB - NB_SLOW
SLOW_CORE = 1
IDXROWS = 2 * 16 * NB
NTILES = 32
BL = 400
GS = 40
PAIRS = 125


def _spmm_sc(a, src_idx, dst_idx, zeros):
    mesh = plsc.VectorSubcoreMesh(core_axis_name="c", subcore_axis_name="s")

    @functools.partial(
        pl.kernel,
        mesh=mesh,
        out_type=jax.ShapeDtypeStruct((2, NPAD, F), jnp.float32),
        scratch_types=[
            pltpu.VMEM((CH, BW), jnp.int32),
            pltpu.VMEM((CH, BW), jnp.int32),
            pltpu.VMEM((BW, F), jnp.float32),
            pltpu.VMEM((BW, F), jnp.float32),
            pltpu.VMEM_SHARED((NPAD, F), jnp.float32),
            pltpu.SemaphoreType.DMA,
            pltpu.SemaphoreType.DMA,
            pltpu.SemaphoreType.DMA,
            pltpu.SemaphoreType.DMA,
        ],
    )
    def k(a_hbm, src_hbm, dst_hbm, z_hbm, out_hbm, src_v, dst_v, rows0, rows1,
          acc, gsem0, gsem1, ssem0, ssem1):
        c = lax.axis_index("c")
        s = lax.axis_index("s")
        r0 = s * RPT
        pltpu.sync_copy(z_hbm.at[pl.ds(r0, RPT)], acc.at[pl.ds(r0, RPT)])
        plsc.subcore_barrier()
        on_slow = c == SLOW_CORE
        base = jnp.where(on_slow, s * NB_SLOW, 16 * NB_SLOW + s * NB_FAST)
        nch = jnp.where(on_slow, NB_SLOW // CH, NB_FAST // CH)

        def chunk(cc, carry):
            pltpu.sync_copy(src_hbm.at[pl.ds(base + cc * CH, CH)], src_v)
            pltpu.sync_copy(dst_hbm.at[pl.ds(base + cc * CH, CH)], dst_v)
            pltpu.async_copy(a_hbm.at[src_v.at[0]], rows0, gsem0)
            pltpu.async_copy(a_hbm.at[src_v.at[1]], rows1, gsem1)

            def m_body(m, carry2):
                j0 = 2 * m
                pltpu.make_async_copy(a_hbm.at[src_v.at[j0]], rows0, gsem0).wait()
                pltpu.async_copy(rows0, acc.at[dst_v.at[j0]], ssem0, add=True)
                pltpu.make_async_copy(a_hbm.at[src_v.at[j0 + 1]], rows1, gsem1).wait()
                pltpu.async_copy(rows1, acc.at[dst_v.at[j0 + 1]], ssem1, add=True)

                @pl.when(m < CH // 2 - 1)
                def _():
                    pltpu.make_async_copy(rows0, acc.at[dst_v.at[j0]], ssem0).wait()
                    pltpu.async_copy(a_hbm.at[src_v.at[j0 + 2]], rows0, gsem0)
                    pltpu.make_async_copy(rows1, acc.at[dst_v.at[j0 + 1]], ssem1).wait()
                    pltpu.async_copy(a_hbm.at[src_v.at[j0 + 3]], rows1, gsem1)

                return carry2

            lax.fori_loop(0, CH // 2, m_body, 0)
            pltpu.make_async_copy(rows0, acc.at[dst_v.at[0]], ssem0).wait()
            pltpu.make_async_copy(rows1, acc.at[dst_v.at[1]], ssem1).wait()
            return carry

        lax.fori_loop(0, nch, chunk, 0)
        plsc.subcore_barrier()
        pltpu.sync_copy(acc.at[pl.ds(r0, RPT)], out_hbm.at[c, pl.ds(r0, RPT)])

    return k(a, src_idx, dst_idx, zeros)


def _node_spec():
    return pl.BlockSpec((BL, F), lambda i: (i, 0))


def _full_spec(shape):
    return pl.BlockSpec(shape, lambda i: tuple(0 for _ in shape))


def _tc_pre(nf, efa0, efa1, wen, ben, wee, bee, wmsg, bmsg):
    f32 = jnp.float32

    def body(nf_r, e0_r, e1_r, wen_r, ben_r, wee_r, bee_r, wmsg_r, bmsg_r,
             h_o, a_o, b_o, kc_o, dg_o):
        h0 = jnp.dot(nf_r[:], wen_r[:], preferred_element_type=f32) + ben_r[:]
        w1 = wmsg_r[0:128, :]
        w2 = wmsg_r[128:256, :]
        w3 = wmsg_r[256:272, :]
        ea = e0_r[:] + e1_r[:]
        we3 = jnp.dot(wee_r[:], w3, preferred_element_type=f32)
        bb = jnp.dot(bee_r[:], w3, preferred_element_type=f32) + bmsg_r[:]
        m_kc = jnp.concatenate([we3, bb, jnp.zeros((111, F), f32)], axis=0)
        m_dg = jnp.concatenate(
            [jnp.zeros((16, F), f32), jnp.ones((1, F), f32), jnp.zeros((111, F), f32)],
            axis=0)
        kc_o[:] = jnp.dot(ea, m_kc, preferred_element_type=f32)
        dg_o[:] = jnp.dot(ea, m_dg, preferred_element_type=f32)
        h_o[:] = h0
        a_o[:] = jnp.dot(h0, w1, preferred_element_type=f32)
        b_o[:] = jnp.dot(h0, w2, preferred_element_type=f32)

    out = jax.ShapeDtypeStruct((N, F), f32)
    return pl.pallas_call(
        body,
        grid=(N // BL,),
        in_specs=[
            _node_spec(),
            _node_spec(),
            _node_spec(),
            _full_spec((F, F)),
            _full_spec((1, F)),
            _full_spec((16, 16)),
            _full_spec((1, 16)),
            _full_spec((272, F)),
            _full_spec((1, F)),
        ],
        out_specs=[_node_spec()] * 5,
        out_shape=[out] * 5,
    )(nf, efa0, efa1, wen, ben, wee, bee, wmsg, bmsg)


def _tc_update(s0, s1, b, h, kc, dg, wupd, bupd, wmsg):
    f32 = jnp.float32

    def body(s0_r, s1_r, b_r, h_r, kc_r, dg_r, wupd_r, bupd_r, wmsg_r,
             h_o, a_o, b_o):
        agg = s0_r[:] + s1_r[:] + kc_r[:] + dg_r[:] * b_r[:]
        hn = (jnp.dot(agg, wupd_r[0:128, :], preferred_element_type=f32)
              + jnp.dot(h_r[:], wupd_r[128:256, :], preferred_element_type=f32)
              + bupd_r[:])
        h_o[:] = hn
        a_o[:] = jnp.dot(hn, wmsg_r[0:128, :], preferred_element_type=f32)
        b_o[:] = jnp.dot(hn, wmsg_r[128:256, :], preferred_element_type=f32)

    out = jax.ShapeDtypeStruct((N, F), f32)
    return pl.pallas_call(
        body,
        grid=(N // BL,),
        in_specs=[
            _node_spec(), _node_spec(), _node_spec(), _node_spec(),
            _node_spec(), _node_spec(),
            _full_spec((256, F)),
            _full_spec((1, F)),
            _full_spec((272, F)),
        ],
        out_specs=[_node_spec()] * 3,
        out_shape=[out] * 3,
    )(s0, s1, b, h, kc, dg, wupd, bupd, wmsg)


def _tc_final(h, wagg, bagg):
    f32 = jnp.float32

    def body(h_r, wagg_r, bagg_r, o_r):
        g = jnp.dot(h_r[:], wagg_r[:], preferred_element_type=f32) + bagg_r[:]
        gates = jax.nn.sigmoid(g[:, 0:F])
        feat = g[:, F:2 * F] * gates
        q = feat[0:GS, :]
        c = feat[GS:2 * GS, :]
        sc = lax.dot_general(q, c, (((1,), (1,)), ((), ())),
                             preferred_element_type=f32)
        tot = jnp.sum(jnp.max(sc, axis=1))
        o_r[:] = jnp.full((1, 8, F), tot, f32)

    return pl.pallas_call(
        body,
        grid=(PAIRS,),
        in_specs=[
            pl.BlockSpec((2 * GS, F), lambda i: (i, 0)),
            _full_spec((F, 2 * F)),
            _full_spec((1, 2 * F)),
        ],
        out_specs=pl.BlockSpec((1, 8, F), lambda i: (i, 0, 0)),
        out_shape=jax.ShapeDtypeStruct((PAIRS, 8, F), f32),
    )(h, wagg, bagg)


def kernel(node_features, edge_features, from_idx, to_idx, graph_idx,
           batch_data_sizes, W_enc_node, b_enc_node, W_enc_edge, b_enc_edge,
           W_msg, b_msg, W_upd, b_upd, W_agg, b_agg):
    f32 = jnp.float32
    i32 = jnp.int32
    ben = b_enc_node.reshape(1, F)
    bee = b_enc_edge.reshape(1, 16)
    bmsg = b_msg.reshape(1, F)
    bupd = b_upd.reshape(1, F)
    bagg = b_agg.reshape(1, 2 * F)

    npair_pad = NTILES * PPT - 2 * E
    src_all = jnp.concatenate(
        [from_idx, to_idx, jnp.zeros((npair_pad,), i32)]).reshape(IDXROWS, BW)
    dst_all = jnp.concatenate(
        [to_idx, from_idx, jnp.full((npair_pad,), DUMMY, i32)]).reshape(IDXROWS, BW)

    eids = jnp.arange(E, dtype=i32)
    esrc = jnp.concatenate(
        [eids, eids, jnp.zeros((npair_pad,), i32)]).reshape(IDXROWS, BW)
    ef128 = jnp.concatenate(
        [edge_features, jnp.ones((E, 1), f32), jnp.zeros((E, F - 17), f32)], axis=1)

    z_f = jnp.zeros((NPAD, F), f32)

    efa = _spmm_sc(ef128, esrc, dst_all, z_f)
    h, a, b, kc, dg = _tc_pre(node_features, efa[0], efa[1], W_enc_node, ben,
                              W_enc_edge, bee, W_msg, bmsg)
    for _ in range(3):
        s = _spmm_sc(a, src_all, dst_all, z_f)
        h, a, b = _tc_update(s[0], s[1], b, h, kc, dg, W_upd, bupd, W_msg)
    out = _tc_final(h, W_agg, bagg)
    return out[:, 0, 0]


# speedup vs baseline: 1.0712x; 1.0452x over previous
import functools

import jax
import jax.numpy as jnp
from jax import lax
from jax.experimental import pallas as pl
from jax.experimental.pallas import tpu as pltpu
from jax.experimental.pallas import tpu_sc as plsc

N = 10000
E = 320000
F = 128
NPAD = 10112
DUMMY = 10008
RPT = NPAD // 16
BW = 128
NB = 160
CH = 32
NCH = NB // CH
PPT = NB * BW
NB_SLOW = 160
NB_FAST = 2 * NB - NB_SLOW
SLOW_CORE = 1
IDXROWS = 2 * 16 * NB
NTILES = 32
BL = 400
GS = 40
PAIRS = 125


def _spmm_sc(a, src_idx, dst_idx, zeros):
    mesh = plsc.VectorSubcoreMesh(core_axis_name="c", subcore_axis_name="s")

    @functools.partial(
        pl.kernel,
        mesh=mesh,
        out_type=jax.ShapeDtypeStruct((2, NPAD, F), jnp.float32),
        scratch_types=[
            pltpu.VMEM((CH, BW), jnp.int32),
            pltpu.VMEM((CH, BW), jnp.int32),
            pltpu.VMEM((BW, F), jnp.float32),
            pltpu.VMEM((BW, F), jnp.float32),
            pltpu.VMEM_SHARED((NPAD, F), jnp.float32),
            pltpu.SemaphoreType.DMA,
            pltpu.SemaphoreType.DMA,
            pltpu.SemaphoreType.DMA,
            pltpu.SemaphoreType.DMA,
        ],
    )
    def k(a_hbm, src_hbm, dst_hbm, z_hbm, out_hbm, src_v, dst_v, rows0, rows1,
          acc, gsem0, gsem1, ssem0, ssem1):
        c = lax.axis_index("c")
        s = lax.axis_index("s")
        r0 = s * RPT
        pltpu.sync_copy(z_hbm.at[pl.ds(r0, RPT)], acc.at[pl.ds(r0, RPT)])
        plsc.subcore_barrier()
        on_slow = c == SLOW_CORE
        base = jnp.where(on_slow, s * NB_SLOW, 16 * NB_SLOW + s * NB_FAST)
        nch = jnp.where(on_slow, NB_SLOW // CH, NB_FAST // CH)

        def chunk(cc, carry):
            pltpu.sync_copy(src_hbm.at[pl.ds(base + cc * CH, CH)], src_v)
            pltpu.sync_copy(dst_hbm.at[pl.ds(base + cc * CH, CH)], dst_v)
            pltpu.async_copy(a_hbm.at[src_v.at[0]], rows0, gsem0)
            pltpu.async_copy(a_hbm.at[src_v.at[1]], rows1, gsem1)

            def m_body(m, carry2):
                j0 = 2 * m
                pltpu.make_async_copy(a_hbm.at[src_v.at[j0]], rows0, gsem0).wait()
                pltpu.async_copy(rows0, acc.at[dst_v.at[j0]], ssem0, add=True)
                pltpu.make_async_copy(a_hbm.at[src_v.at[j0 + 1]], rows1, gsem1).wait()
                pltpu.async_copy(rows1, acc.at[dst_v.at[j0 + 1]], ssem1, add=True)

                @pl.when(m < CH // 2 - 1)
                def _():
                    pltpu.make_async_copy(rows0, acc.at[dst_v.at[j0]], ssem0).wait()
                    pltpu.async_copy(a_hbm.at[src_v.at[j0 + 2]], rows0, gsem0)
                    pltpu.make_async_copy(rows1, acc.at[dst_v.at[j0 + 1]], ssem1).wait()
                    pltpu.async_copy(a_hbm.at[src_v.at[j0 + 3]], rows1, gsem1)

                return carry2

            lax.fori_loop(0, CH // 2, m_body, 0)
            pltpu.make_async_copy(rows0, acc.at[dst_v.at[0]], ssem0).wait()
            pltpu.make_async_copy(rows1, acc.at[dst_v.at[1]], ssem1).wait()
            return carry

        lax.fori_loop(0, nch, chunk, 0)
        plsc.subcore_barrier()
        pltpu.sync_copy(acc.at[pl.ds(r0, RPT)], out_hbm.at[c, pl.ds(r0, RPT)])

    return k(a, src_idx, dst_idx, zeros)


def _node_spec():
    return pl.BlockSpec((BL, F), lambda i: (i, 0))


def _full_spec(shape):
    return pl.BlockSpec(shape, lambda i: tuple(0 for _ in shape))


def _tc_pre(nf, wen, ben, wmsg):
    f32 = jnp.float32

    def body(nf_r, wen_r, ben_r, wmsg_r, h_o, a_o, b_o):
        h0 = jnp.dot(nf_r[:], wen_r[:], preferred_element_type=f32) + ben_r[:]
        h_o[:] = h0
        a_o[:] = jnp.dot(h0, wmsg_r[0:128, :], preferred_element_type=f32)
        b_o[:] = jnp.dot(h0, wmsg_r[128:256, :], preferred_element_type=f32)

    out = jax.ShapeDtypeStruct((N, F), f32)
    return pl.pallas_call(
        body,
        grid=(N // BL,),
        in_specs=[
            _node_spec(),
            _full_spec((F, F)),
            _full_spec((1, F)),
            _full_spec((272, F)),
        ],
        out_specs=[_node_spec()] * 3,
        out_shape=[out] * 3,
    )(nf, wen, ben, wmsg)


def _tc_update(s0, s1, b, h, efa0, efa1, wupd, bupd, wmsg, wee, bee, bmsg):
    f32 = jnp.float32

    def body(s0_r, s1_r, b_r, h_r, e0_r, e1_r, wupd_r, bupd_r, wmsg_r,
             wee_r, bee_r, bmsg_r, h_o, a_o, b_o):
        ea = e0_r[:] + e1_r[:]
        w3 = wmsg_r[256:272, :]
        we3 = jnp.dot(wee_r[:], w3, preferred_element_type=f32)
        bb = jnp.dot(bee_r[:], w3, preferred_element_type=f32) + bmsg_r[:]
        m_kc = jnp.concatenate([we3, bb, jnp.zeros((111, F), f32)], axis=0)
        m_dg = jnp.concatenate(
            [jnp.zeros((16, F), f32), jnp.ones((1, F), f32), jnp.zeros((111, F), f32)],
            axis=0)
        kc = jnp.dot(ea, m_kc, preferred_element_type=f32)
        dg = jnp.dot(ea, m_dg, preferred_element_type=f32)
        agg = s0_r[:] + s1_r[:] + kc + dg * b_r[:]
        hn = (jnp.dot(agg, wupd_r[0:128, :], preferred_element_type=f32)
              + jnp.dot(h_r[:], wupd_r[128:256, :], preferred_element_type=f32)
              + bupd_r[:])
        h_o[:] = hn
        a_o[:] = jnp.dot(hn, wmsg_r[0:128, :], preferred_element_type=f32)
        b_o[:] = jnp.dot(hn, wmsg_r[128:256, :], preferred_element_type=f32)

    out = jax.ShapeDtypeStruct((N, F), f32)
    return pl.pallas_call(
        body,
        grid=(N // BL,),
        in_specs=[
            _node_spec(), _node_spec(), _node_spec(), _node_spec(),
            _node_spec(), _node_spec(),
            _full_spec((256, F)),
            _full_spec((1, F)),
            _full_spec((272, F)),
            _full_spec((16, 16)),
            _full_spec((1, 16)),
            _full_spec((1, F)),
        ],
        out_specs=[_node_spec()] * 3,
        out_shape=[out] * 3,
    )(s0, s1, b, h, efa0, efa1, wupd, bupd, wmsg, wee, bee, bmsg)


def _tc_final(h, wagg, bagg):
    f32 = jnp.float32

    def body(h_r, wagg_r, bagg_r, o_r):
        g = jnp.dot(h_r[:], wagg_r[:], preferred_element_type=f32) + bagg_r[:]
        gates = jax.nn.sigmoid(g[:, 0:F])
        feat = g[:, F:2 * F] * gates
        q = feat[0:GS, :]
        c = feat[GS:2 * GS, :]
        sc = lax.dot_general(q, c, (((1,), (1,)), ((), ())),
                             preferred_element_type=f32)
        tot = jnp.sum(jnp.max(sc, axis=1))
        o_r[:] = jnp.full((1, 8, F), tot, f32)

    return pl.pallas_call(
        body,
        grid=(PAIRS,),
        in_specs=[
            pl.BlockSpec((2 * GS, F), lambda i: (i, 0)),
            _full_spec((F, 2 * F)),
            _full_spec((1, 2 * F)),
        ],
        out_specs=pl.BlockSpec((1, 8, F), lambda i: (i, 0, 0)),
        out_shape=jax.ShapeDtypeStruct((PAIRS, 8, F), f32),
    )(h, wagg, bagg)


def kernel(node_features, edge_features, from_idx, to_idx, graph_idx,
           batch_data_sizes, W_enc_node, b_enc_node, W_enc_edge, b_enc_edge,
           W_msg, b_msg, W_upd, b_upd, W_agg, b_agg):
    f32 = jnp.float32
    i32 = jnp.int32
    ben = b_enc_node.reshape(1, F)
    bee = b_enc_edge.reshape(1, 16)
    bmsg = b_msg.reshape(1, F)
    bupd = b_upd.reshape(1, F)
    bagg = b_agg.reshape(1, 2 * F)

    npair_pad = NTILES * PPT - 2 * E
    src_all = jnp.concatenate(
        [from_idx, to_idx, jnp.zeros((npair_pad,), i32)]).reshape(IDXROWS, BW)
    dst_all = jnp.concatenate(
        [to_idx, from_idx, jnp.full((npair_pad,), DUMMY, i32)]).reshape(IDXROWS, BW)

    eids = jnp.arange(E, dtype=i32)
    esrc = jnp.concatenate(
        [eids, eids, jnp.zeros((npair_pad,), i32)]).reshape(IDXROWS, BW)
    ef128 = jnp.concatenate(
        [edge_features, jnp.ones((E, 1), f32), jnp.zeros((E, F - 17), f32)], axis=1)

    z_f = jnp.zeros((NPAD, F), f32)

    efa = _spmm_sc(ef128, esrc, dst_all, z_f)
    h, a, b = _tc_pre(node_features, W_enc_node, ben, W_msg)
    for _ in range(3):
        s = _spmm_sc(a, src_all, dst_all, z_f)
        h, a, b = _tc_update(s[0], s[1], b, h, efa[0], efa[1], W_upd, bupd,
                             W_msg, W_enc_edge, bee, bmsg)
    out = _tc_final(h, W_agg, bagg)
    return out[:, 0, 0]


# speedup vs baseline: 1.0750x; 1.0035x over previous
import functools

import jax
import jax.numpy as jnp
from jax import lax
from jax.experimental import pallas as pl
from jax.experimental.pallas import tpu as pltpu
from jax.experimental.pallas import tpu_sc as plsc

N = 10000
E = 320000
F = 128
NPAD = 10112
DUMMY = 10008
RPT = NPAD // 16
BW = 128
NB = 160
CH = 32
NCH = NB // CH
PPT = NB * BW
IDXROWS = 2 * 16 * NB
NTILES = 32
BL = 400
GS = 40
PAIRS = 125


@functools.lru_cache(maxsize=None)
def _gs_kernel(t_rows):
    mesh = plsc.VectorSubcoreMesh(core_axis_name="c", subcore_axis_name="s")

    @functools.partial(
        pl.kernel,
        mesh=mesh,
        out_type=jax.ShapeDtypeStruct((2, NPAD, F), jnp.float32),
        scratch_types=[
            pltpu.VMEM((CH, BW), jnp.int32),
            pltpu.VMEM((CH, BW), jnp.int32),
            pltpu.VMEM((BW, F), jnp.float32),
            pltpu.VMEM((BW, F), jnp.float32),
            pltpu.VMEM_SHARED((NPAD, F), jnp.float32),
            pltpu.SemaphoreType.DMA,
            pltpu.SemaphoreType.DMA,
            pltpu.SemaphoreType.DMA,
            pltpu.SemaphoreType.DMA,
        ],
    )
    def k(a_hbm, src_hbm, dst_hbm, z_hbm, out_hbm, src_v, dst_v, rows0, rows1,
          acc, gsem0, gsem1, ssem0, ssem1):
        c = lax.axis_index("c")
        s = lax.axis_index("s")
        r0 = s * RPT
        pltpu.sync_copy(z_hbm.at[pl.ds(r0, RPT)], acc.at[pl.ds(r0, RPT)])
        plsc.subcore_barrier()
        base = (c * 16 + s) * NB
        nch = NCH

        def chunk(cc, carry):
            pltpu.sync_copy(src_hbm.at[pl.ds(base + cc * CH, CH)], src_v)
            pltpu.sync_copy(dst_hbm.at[pl.ds(base + cc * CH, CH)], dst_v)
            pltpu.async_copy(a_hbm.at[src_v.at[0]], rows0, gsem0)
            pltpu.async_copy(a_hbm.at[src_v.at[1]], rows1, gsem1)

            def m_body(m, carry2):
                j0 = 2 * m
                pltpu.make_async_copy(a_hbm.at[src_v.at[j0]], rows0, gsem0).wait()
                pltpu.async_copy(rows0, acc.at[dst_v.at[j0]], ssem0, add=True)
                pltpu.make_async_copy(a_hbm.at[src_v.at[j0 + 1]], rows1, gsem1).wait()
                pltpu.async_copy(rows1, acc.at[dst_v.at[j0 + 1]], ssem1, add=True)

                @pl.when(m < CH // 2 - 1)
                def _():
                    pltpu.make_async_copy(rows0, acc.at[dst_v.at[j0]], ssem0).wait()
                    pltpu.async_copy(a_hbm.at[src_v.at[j0 + 2]], rows0, gsem0)
                    pltpu.make_async_copy(rows1, acc.at[dst_v.at[j0 + 1]], ssem1).wait()
                    pltpu.async_copy(a_hbm.at[src_v.at[j0 + 3]], rows1, gsem1)

                return carry2

            lax.fori_loop(0, CH // 2, m_body, 0)
            pltpu.make_async_copy(rows0, acc.at[dst_v.at[0]], ssem0).wait()
            pltpu.make_async_copy(rows1, acc.at[dst_v.at[1]], ssem1).wait()
            return carry

        lax.fori_loop(0, nch, chunk, 0)
        plsc.subcore_barrier()
        pltpu.sync_copy(acc.at[pl.ds(r0, RPT)], out_hbm.at[c, pl.ds(r0, RPT)])

    return k


def _spmm_sc(a, src_idx, dst_idx, zeros):
    return _gs_kernel(a.shape[0])(a, src_idx, dst_idx, zeros)


def _node_spec():
    return pl.BlockSpec((BL, F), lambda i: (i, 0))


def _full_spec(shape):
    return pl.BlockSpec(shape, lambda i: tuple(0 for _ in shape))


def _tc_pre(nf, wen, ben, wmsg):
    f32 = jnp.float32

    def body(nf_r, wen_r, ben_r, wmsg_r, h_o, a_o, b_o):
        h0 = jnp.dot(nf_r[:], wen_r[:], preferred_element_type=f32) + ben_r[:]
        h_o[:] = h0
        a_o[:] = jnp.dot(h0, wmsg_r[0:128, :], preferred_element_type=f32)
        b_o[:] = jnp.dot(h0, wmsg_r[128:256, :], preferred_element_type=f32)

    out = jax.ShapeDtypeStruct((N, F), f32)
    return pl.pallas_call(
        body,
        grid=(N // BL,),
        in_specs=[
            _node_spec(),
            _full_spec((F, F)),
            _full_spec((1, F)),
            _full_spec((272, F)),
        ],
        out_specs=[_node_spec()] * 3,
        out_shape=[out] * 3,
    )(nf, wen, ben, wmsg)


def _tc_update(s0, s1, b, h, efa0, efa1, wupd, bupd, wmsg, wee, bee, bmsg):
    f32 = jnp.float32

    def body(s0_r, s1_r, b_r, h_r, e0_r, e1_r, wupd_r, bupd_r, wmsg_r,
             wee_r, bee_r, bmsg_r, h_o, a_o, b_o):
        ea = e0_r[:] + e1_r[:]
        w3 = wmsg_r[256:272, :]
        we3 = jnp.dot(wee_r[:], w3, preferred_element_type=f32)
        bb = jnp.dot(bee_r[:], w3, preferred_element_type=f32) + bmsg_r[:]
        m_kc = jnp.concatenate([we3, bb, jnp.zeros((111, F), f32)], axis=0)
        m_dg = jnp.concatenate(
            [jnp.zeros((16, F), f32), jnp.ones((1, F), f32), jnp.zeros((111, F), f32)],
            axis=0)
        kc = jnp.dot(ea, m_kc, preferred_element_type=f32)
        dg = jnp.dot(ea, m_dg, preferred_element_type=f32)
        agg = s0_r[:] + s1_r[:] + kc + dg * b_r[:]
        hn = (jnp.dot(agg, wupd_r[0:128, :], preferred_element_type=f32)
              + jnp.dot(h_r[:], wupd_r[128:256, :], preferred_element_type=f32)
              + bupd_r[:])
        h_o[:] = hn
        a_o[:] = jnp.dot(hn, wmsg_r[0:128, :], preferred_element_type=f32)
        b_o[:] = jnp.dot(hn, wmsg_r[128:256, :], preferred_element_type=f32)

    out = jax.ShapeDtypeStruct((N, F), f32)
    return pl.pallas_call(
        body,
        grid=(N // BL,),
        in_specs=[
            _node_spec(), _node_spec(), _node_spec(), _node_spec(),
            _node_spec(), _node_spec(),
            _full_spec((256, F)),
            _full_spec((1, F)),
            _full_spec((272, F)),
            _full_spec((16, 16)),
            _full_spec((1, 16)),
            _full_spec((1, F)),
        ],
        out_specs=[_node_spec()] * 3,
        out_shape=[out] * 3,
    )(s0, s1, b, h, efa0, efa1, wupd, bupd, wmsg, wee, bee, bmsg)


def _tc_final(h, wagg, bagg):
    f32 = jnp.float32

    def body(h_r, wagg_r, bagg_r, o_r):
        g = jnp.dot(h_r[:], wagg_r[:], preferred_element_type=f32) + bagg_r[:]
        gates = jax.nn.sigmoid(g[:, 0:F])
        feat = g[:, F:2 * F] * gates
        q = feat[0:GS, :]
        c = feat[GS:2 * GS, :]
        sc = lax.dot_general(q, c, (((1,), (1,)), ((), ())),
                             preferred_element_type=f32)
        tot = jnp.sum(jnp.max(sc, axis=1))
        o_r[:] = jnp.full((1, 8, F), tot, f32)

    return pl.pallas_call(
        body,
        grid=(PAIRS,),
        in_specs=[
            pl.BlockSpec((2 * GS, F), lambda i: (i, 0)),
            _full_spec((F, 2 * F)),
            _full_spec((1, 2 * F)),
        ],
        out_specs=pl.BlockSpec((1, 8, F), lambda i: (i, 0, 0)),
        out_shape=jax.ShapeDtypeStruct((PAIRS, 8, F), f32),
    )(h, wagg, bagg)


def kernel(node_features, edge_features, from_idx, to_idx, graph_idx,
           batch_data_sizes, W_enc_node, b_enc_node, W_enc_edge, b_enc_edge,
           W_msg, b_msg, W_upd, b_upd, W_agg, b_agg):
    f32 = jnp.float32
    i32 = jnp.int32
    ben = b_enc_node.reshape(1, F)
    bee = b_enc_edge.reshape(1, 16)
    bmsg = b_msg.reshape(1, F)
    bupd = b_upd.reshape(1, F)
    bagg = b_agg.reshape(1, 2 * F)

    npair_pad = NTILES * PPT - 2 * E
    src_all = jnp.concatenate(
        [from_idx, to_idx, jnp.zeros((npair_pad,), i32)]).reshape(IDXROWS, BW)
    dst_all = jnp.concatenate(
        [to_idx, from_idx, jnp.full((npair_pad,), DUMMY, i32)]).reshape(IDXROWS, BW)

    eids = jnp.arange(E, dtype=i32)
    esrc = jnp.concatenate(
        [eids, eids, jnp.zeros((npair_pad,), i32)]).reshape(IDXROWS, BW)
    ef128 = jnp.concatenate(
        [edge_features, jnp.ones((E, 1), f32), jnp.zeros((E, F - 17), f32)], axis=1)

    z_f = jnp.zeros((NPAD, F), f32)

    efa = _spmm_sc(ef128, esrc, dst_all, z_f)
    h, a, b = _tc_pre(node_features, W_enc_node, ben, W_msg)
    for _ in range(3):
        s = _spmm_sc(a, src_all, dst_all, z_f)
        h, a, b = _tc_update(s[0], s[1], b, h, efa[0], efa[1], W_upd, bupd,
                             W_msg, W_enc_edge, bee, bmsg)
    out = _tc_final(h, W_agg, bagg)
    return out[:, 0, 0]


# speedup vs baseline: 1.1049x; 1.0278x over previous
import functools

import jax
import jax.numpy as jnp
from jax import lax
from jax.experimental import pallas as pl
from jax.experimental.pallas import tpu as pltpu
from jax.experimental.pallas import tpu_sc as plsc

N = 10000
E = 320000
F = 128
NPAD = 10112
DUMMY = 10008
RPT = NPAD // 16
BW = 128
NB = 160
CH = 32
NCH = NB // CH
PPT = NB * BW
IDXROWS = 2 * 16 * NB
NTILES = 32
BL = 400
GS = 40
PAIRS = 125


@functools.lru_cache(maxsize=None)
def _gs_kernel(t_rows):
    mesh = plsc.VectorSubcoreMesh(core_axis_name="c", subcore_axis_name="s")

    @functools.partial(
        pl.kernel,
        mesh=mesh,
        out_type=jax.ShapeDtypeStruct((2, NPAD, F), jnp.float32),
        scratch_types=[
            pltpu.VMEM((CH, BW), jnp.int32),
            pltpu.VMEM((CH, BW), jnp.int32),
            pltpu.VMEM((BW, F), jnp.float32),
            pltpu.VMEM((BW, F), jnp.float32),
            pltpu.VMEM_SHARED((NPAD, F), jnp.float32),
            pltpu.SemaphoreType.DMA,
            pltpu.SemaphoreType.DMA,
            pltpu.SemaphoreType.DMA,
            pltpu.SemaphoreType.DMA,
        ],
    )
    def k(a_hbm, src_hbm, dst_hbm, z_hbm, out_hbm, src_v, dst_v, rows0, rows1,
          acc, gsem0, gsem1, ssem0, ssem1):
        c = lax.axis_index("c")
        s = lax.axis_index("s")
        r0 = s * RPT
        pltpu.sync_copy(z_hbm.at[pl.ds(r0, RPT)], acc.at[pl.ds(r0, RPT)])
        plsc.subcore_barrier()
        base = (c * 16 + s) * NB
        nch = NCH

        def chunk(cc, carry):
            pltpu.sync_copy(src_hbm.at[pl.ds(base + cc * CH, CH)], src_v)
            pltpu.sync_copy(dst_hbm.at[pl.ds(base + cc * CH, CH)], dst_v)
            pltpu.async_copy(a_hbm.at[src_v.at[0]], rows0, gsem0)
            pltpu.async_copy(a_hbm.at[src_v.at[1]], rows1, gsem1)

            def m_body(m, carry2):
                j0 = 2 * m
                pltpu.make_async_copy(a_hbm.at[src_v.at[j0]], rows0, gsem0).wait()
                pltpu.async_copy(rows0, acc.at[dst_v.at[j0]], ssem0, add=True)
                pltpu.make_async_copy(a_hbm.at[src_v.at[j0 + 1]], rows1, gsem1).wait()
                pltpu.async_copy(rows1, acc.at[dst_v.at[j0 + 1]], ssem1, add=True)

                @pl.when(m < CH // 2 - 1)
                def _():
                    pltpu.make_async_copy(rows0, acc.at[dst_v.at[j0]], ssem0).wait()
                    pltpu.async_copy(a_hbm.at[src_v.at[j0 + 2]], rows0, gsem0)
                    pltpu.make_async_copy(rows1, acc.at[dst_v.at[j0 + 1]], ssem1).wait()
                    pltpu.async_copy(a_hbm.at[src_v.at[j0 + 3]], rows1, gsem1)

                return carry2

            lax.fori_loop(0, CH // 2, m_body, 0)
            pltpu.make_async_copy(rows0, acc.at[dst_v.at[0]], ssem0).wait()
            pltpu.make_async_copy(rows1, acc.at[dst_v.at[1]], ssem1).wait()
            return carry

        lax.fori_loop(0, nch, chunk, 0)
        plsc.subcore_barrier()
        pltpu.sync_copy(acc.at[pl.ds(r0, RPT)], out_hbm.at[c, pl.ds(r0, RPT)])

    return k


def _spmm_sc(a, src_idx, dst_idx, zeros):
    return _gs_kernel(a.shape[0])(a, src_idx, dst_idx, zeros)


def _node_spec():
    return pl.BlockSpec((BL, F), lambda i: (i, 0))


def _full_spec(shape):
    return pl.BlockSpec(shape, lambda i: tuple(0 for _ in shape))


def _tc_pre(nf, efa0, efa1, wen, ben, wee, bee, wmsg, bmsg):
    f32 = jnp.float32

    def body(nf_r, e0_r, e1_r, wen_r, ben_r, wee_r, bee_r, wmsg_r, bmsg_r,
             h_o, a_o, b_o, kc_o, dg_o):
        h0 = jnp.dot(nf_r[:], wen_r[:], preferred_element_type=f32) + ben_r[:]
        w3 = wmsg_r[256:272, :]
        ea = e0_r[:] + e1_r[:]
        we3 = jnp.dot(wee_r[:], w3, preferred_element_type=f32)
        bb = jnp.dot(bee_r[:], w3, preferred_element_type=f32) + bmsg_r[:]
        m_kc = jnp.concatenate([we3, bb, jnp.zeros((111, F), f32)], axis=0)
        m_dg = jnp.concatenate(
            [jnp.zeros((16, F), f32), jnp.ones((1, F), f32), jnp.zeros((111, F), f32)],
            axis=0)
        kc_o[:] = jnp.dot(ea, m_kc, preferred_element_type=f32)
        dg_o[:] = jnp.dot(ea, m_dg, preferred_element_type=f32)
        h_o[:] = h0
        a_o[:] = jnp.dot(h0, wmsg_r[0:128, :], preferred_element_type=f32)
        b_o[:] = jnp.dot(h0, wmsg_r[128:256, :], preferred_element_type=f32)

    out = jax.ShapeDtypeStruct((N, F), f32)
    return pl.pallas_call(
        body,
        grid=(N // BL,),
        in_specs=[
            _node_spec(),
            _node_spec(),
            _node_spec(),
            _full_spec((F, F)),
            _full_spec((1, F)),
            _full_spec((16, 16)),
            _full_spec((1, 16)),
            _full_spec((272, F)),
            _full_spec((1, F)),
        ],
        out_specs=[_node_spec()] * 5,
        out_shape=[out] * 5,
    )(nf, efa0, efa1, wen, ben, wee, bee, wmsg, bmsg)


def _tc_update(s0, s1, b, h, kc, dg, wupd, bupd, wmsg):
    f32 = jnp.float32

    def body(s0_r, s1_r, b_r, h_r, kc_r, dg_r, wupd_r, bupd_r, wmsg_r,
             h_o, a_o, b_o):
        agg = s0_r[:] + s1_r[:] + kc_r[:] + dg_r[:] * b_r[:]
        hn = (jnp.dot(agg, wupd_r[0:128, :], preferred_element_type=f32)
              + jnp.dot(h_r[:], wupd_r[128:256, :], preferred_element_type=f32)
              + bupd_r[:])
        h_o[:] = hn
        a_o[:] = jnp.dot(hn, wmsg_r[0:128, :], preferred_element_type=f32)
        b_o[:] = jnp.dot(hn, wmsg_r[128:256, :], preferred_element_type=f32)

    out = jax.ShapeDtypeStruct((N, F), f32)
    return pl.pallas_call(
        body,
        grid=(N // BL,),
        in_specs=[
            _node_spec(), _node_spec(), _node_spec(), _node_spec(),
            _node_spec(), _node_spec(),
            _full_spec((256, F)),
            _full_spec((1, F)),
            _full_spec((272, F)),
        ],
        out_specs=[_node_spec()] * 3,
        out_shape=[out] * 3,
    )(s0, s1, b, h, kc, dg, wupd, bupd, wmsg)


def _tc_final(h, wagg, bagg):
    f32 = jnp.float32

    def body(h_r, wagg_r, bagg_r, o_r):
        g = jnp.dot(h_r[:], wagg_r[:], preferred_element_type=f32) + bagg_r[:]
        gates = jax.nn.sigmoid(g[:, 0:F])
        feat = g[:, F:2 * F] * gates
        q = feat[0:GS, :]
        c = feat[GS:2 * GS, :]
        sc = lax.dot_general(q, c, (((1,), (1,)), ((), ())),
                             preferred_element_type=f32)
        tot = jnp.sum(jnp.max(sc, axis=1))
        o_r[:] = jnp.full((1, 8, F), tot, f32)

    return pl.pallas_call(
        body,
        grid=(PAIRS,),
        in_specs=[
            pl.BlockSpec((2 * GS, F), lambda i: (i, 0)),
            _full_spec((F, 2 * F)),
            _full_spec((1, 2 * F)),
        ],
        out_specs=pl.BlockSpec((1, 8, F), lambda i: (i, 0, 0)),
        out_shape=jax.ShapeDtypeStruct((PAIRS, 8, F), f32),
    )(h, wagg, bagg)


def kernel(node_features, edge_features, from_idx, to_idx, graph_idx,
           batch_data_sizes, W_enc_node, b_enc_node, W_enc_edge, b_enc_edge,
           W_msg, b_msg, W_upd, b_upd, W_agg, b_agg):
    f32 = jnp.float32
    i32 = jnp.int32
    ben = b_enc_node.reshape(1, F)
    bee = b_enc_edge.reshape(1, 16)
    bmsg = b_msg.reshape(1, F)
    bupd = b_upd.reshape(1, F)
    bagg = b_agg.reshape(1, 2 * F)

    npair_pad = NTILES * PPT - 2 * E
    src_all = jnp.concatenate(
        [from_idx, to_idx, jnp.zeros((npair_pad,), i32)]).reshape(IDXROWS, BW)
    dst_all = jnp.concatenate(
        [to_idx, from_idx, jnp.full((npair_pad,), DUMMY, i32)]).reshape(IDXROWS, BW)

    eids = jnp.arange(E, dtype=i32)
    esrc = jnp.concatenate(
        [eids, eids, jnp.zeros((npair_pad,), i32)]).reshape(IDXROWS, BW)
    ef128 = jnp.concatenate(
        [edge_features, jnp.ones((E, 1), f32), jnp.zeros((E, F - 17), f32)], axis=1)

    z_f = jnp.zeros((NPAD, F), f32)

    efa = _spmm_sc(ef128, esrc, dst_all, z_f)
    h, a, b, kc, dg = _tc_pre(node_features, efa[0], efa[1], W_enc_node, ben,
                              W_enc_edge, bee, W_msg, bmsg)
    for _ in range(3):
        s = _spmm_sc(a, src_all, dst_all, z_f)
        h, a, b = _tc_update(s[0], s[1], b, h, kc, dg, W_upd, bupd, W_msg)
    out = _tc_final(h, W_agg, bagg)
    return out[:, 0, 0]


# speedup vs baseline: 1.2115x; 1.0966x over previous
import functools

import jax
import jax.numpy as jnp
from jax import lax
from jax.experimental import pallas as pl
from jax.experimental.pallas import tpu as pltpu
from jax.experimental.pallas import tpu_sc as plsc

N = 10000
E = 320000
F = 128
NPAD = 10112
DUMMY = 10008
RPT = NPAD // 16
BW = 128
NB = 160
CH = 32
NCH = NB // CH
PPT = NB * BW
IDXROWS = 2 * 16 * NB
NTILES = 32
BL = 400
GS = 40
PAIRS = 125


@functools.lru_cache(maxsize=None)
def _gs_kernel(t_rows):
    mesh = plsc.VectorSubcoreMesh(core_axis_name="c", subcore_axis_name="s")

    @functools.partial(
        pl.kernel,
        mesh=mesh,
        out_type=jax.ShapeDtypeStruct((2, NPAD, F), jnp.float32),
        scratch_types=[
            pltpu.VMEM((CH, BW), jnp.int32),
            pltpu.VMEM((CH, BW), jnp.int32),
            pltpu.VMEM((BW, F), jnp.float32),
            pltpu.VMEM((BW, F), jnp.float32),
            pltpu.VMEM_SHARED((NPAD, F), jnp.float32),
            pltpu.SemaphoreType.DMA,
            pltpu.SemaphoreType.DMA,
            pltpu.SemaphoreType.DMA,
            pltpu.SemaphoreType.DMA,
        ],
    )
    def k(a_hbm, src_hbm, dst_hbm, z_hbm, out_hbm, src_v, dst_v, rows0, rows1,
          acc, gsem0, gsem1, ssem0, ssem1):
        c = lax.axis_index("c")
        s = lax.axis_index("s")
        r0 = s * RPT
        pltpu.sync_copy(z_hbm.at[pl.ds(r0, RPT)], acc.at[pl.ds(r0, RPT)])
        plsc.subcore_barrier()
        base = (c * 16 + s) * NB
        nch = NCH

        def chunk(cc, carry):
            pltpu.sync_copy(src_hbm.at[pl.ds(base + cc * CH, CH)], src_v)
            pltpu.sync_copy(dst_hbm.at[pl.ds(base + cc * CH, CH)], dst_v)
            pltpu.async_copy(a_hbm.at[src_v.at[0]], rows0, gsem0)
            pltpu.async_copy(a_hbm.at[src_v.at[1]], rows1, gsem1)

            def m_body(m, carry2):
                j0 = 2 * m
                pltpu.make_async_copy(a_hbm.at[src_v.at[j0]], rows0, gsem0).wait()
                pltpu.async_copy(rows0, acc.at[dst_v.at[j0]], ssem0, add=True)
                pltpu.make_async_copy(a_hbm.at[src_v.at[j0 + 1]], rows1, gsem1).wait()
                pltpu.async_copy(rows1, acc.at[dst_v.at[j0 + 1]], ssem1, add=True)

                @pl.when(m < CH // 2 - 1)
                def _():
                    pltpu.make_async_copy(rows0, acc.at[dst_v.at[j0]], ssem0).wait()
                    pltpu.async_copy(a_hbm.at[src_v.at[j0 + 2]], rows0, gsem0)
                    pltpu.make_async_copy(rows1, acc.at[dst_v.at[j0 + 1]], ssem1).wait()
                    pltpu.async_copy(a_hbm.at[src_v.at[j0 + 3]], rows1, gsem1)

                return carry2

            lax.fori_loop(0, CH // 2, m_body, 0)
            pltpu.make_async_copy(rows0, acc.at[dst_v.at[0]], ssem0).wait()
            pltpu.make_async_copy(rows1, acc.at[dst_v.at[1]], ssem1).wait()
            return carry

        lax.fori_loop(0, nch, chunk, 0)
        plsc.subcore_barrier()
        pltpu.sync_copy(acc.at[pl.ds(r0, RPT)], out_hbm.at[c, pl.ds(r0, RPT)])

    return k


def _spmm_sc(a, src_idx, dst_idx, zeros):
    return _gs_kernel(a.shape[0])(a, src_idx, dst_idx, zeros)


def _node_spec():
    return pl.BlockSpec((BL, F), lambda i: (i, 0))


def _full_spec(shape):
    return pl.BlockSpec(shape, lambda i: tuple(0 for _ in shape))


def _tc_pre(nf, efa0, efa1, wen, ben, wee, bee, wmsg, bmsg):
    f32 = jnp.float32

    def body(nf_r, e0_r, e1_r, wen_r, ben_r, wee_r, bee_r, wmsg_r, bmsg_r,
             h_o, a_o, b_o, kc_o, dg_o):
        h0 = jnp.dot(nf_r[:], wen_r[:], preferred_element_type=f32) + ben_r[:]
        w3 = wmsg_r[256:272, :]
        ea = e0_r[:] + e1_r[:]
        we3 = jnp.dot(wee_r[:], w3, preferred_element_type=f32)
        bb = jnp.dot(bee_r[:], w3, preferred_element_type=f32) + bmsg_r[:]
        m_kc = jnp.concatenate([we3, bb, jnp.zeros((111, F), f32)], axis=0)
        m_dg = jnp.concatenate(
            [jnp.zeros((16, F), f32), jnp.ones((1, F), f32), jnp.zeros((111, F), f32)],
            axis=0)
        kc_o[:] = jnp.dot(ea, m_kc, preferred_element_type=f32)
        dg_o[:] = jnp.dot(ea, m_dg, preferred_element_type=f32)
        h_o[:] = h0
        a_o[:] = jnp.dot(h0, wmsg_r[0:128, :], preferred_element_type=f32)
        b_o[:] = jnp.dot(h0, wmsg_r[128:256, :], preferred_element_type=f32)

    out = jax.ShapeDtypeStruct((N, F), f32)
    return pl.pallas_call(
        body,
        grid=(N // BL,),
        in_specs=[
            _node_spec(),
            _node_spec(),
            _node_spec(),
            _full_spec((F, F)),
            _full_spec((1, F)),
            _full_spec((16, 16)),
            _full_spec((1, 16)),
            _full_spec((272, F)),
            _full_spec((1, F)),
        ],
        out_specs=[_node_spec()] * 5,
        out_shape=[out] * 5,
    )(nf, efa0, efa1, wen, ben, wee, bee, wmsg, bmsg)


def _tc_update(s0, s1, b, h, kc, dg, wupd, bupd, wmsg):
    f32 = jnp.float32

    def body(s0_r, s1_r, b_r, h_r, kc_r, dg_r, wupd_r, bupd_r, wmsg_r,
             h_o, a_o, b_o):
        agg = s0_r[:] + s1_r[:] + kc_r[:] + dg_r[:] * b_r[:]
        hn = (jnp.dot(agg, wupd_r[0:128, :], preferred_element_type=f32)
              + jnp.dot(h_r[:], wupd_r[128:256, :], preferred_element_type=f32)
              + bupd_r[:])
        h_o[:] = hn
        a_o[:] = jnp.dot(hn, wmsg_r[0:128, :], preferred_element_type=f32)
        b_o[:] = jnp.dot(hn, wmsg_r[128:256, :], preferred_element_type=f32)

    out = jax.ShapeDtypeStruct((N, F), f32)
    return pl.pallas_call(
        body,
        grid=(N // BL,),
        in_specs=[
            _node_spec(), _node_spec(), _node_spec(), _node_spec(),
            _node_spec(), _node_spec(),
            _full_spec((256, F)),
            _full_spec((1, F)),
            _full_spec((272, F)),
        ],
        out_specs=[_node_spec()] * 3,
        out_shape=[out] * 3,
    )(s0, s1, b, h, kc, dg, wupd, bupd, wmsg)


def _tc_update_final(s0, s1, b, h, kc, dg, wupd, bupd, wagg, bagg):
    f32 = jnp.float32
    ppb = BL // (2 * GS)

    def body(s0_r, s1_r, b_r, h_r, kc_r, dg_r, wupd_r, bupd_r, wagg_r, bagg_r,
             o_r):
        agg = s0_r[:] + s1_r[:] + kc_r[:] + dg_r[:] * b_r[:]
        hn = (jnp.dot(agg, wupd_r[0:128, :], preferred_element_type=f32)
              + jnp.dot(h_r[:], wupd_r[128:256, :], preferred_element_type=f32)
              + bupd_r[:])
        g = jnp.dot(hn, wagg_r[:], preferred_element_type=f32) + bagg_r[:]
        gates = jax.nn.sigmoid(g[:, 0:F])
        feat = g[:, F:2 * F] * gates
        for p in range(ppb):
            q = feat[2 * GS * p:2 * GS * p + GS, :]
            c = feat[2 * GS * p + GS:2 * GS * (p + 1), :]
            sc = lax.dot_general(q, c, (((1,), (1,)), ((), ())),
                                 preferred_element_type=f32)
            tot = jnp.sum(jnp.max(sc, axis=1))
            o_r[p] = jnp.full((8, F), tot, f32)

    return pl.pallas_call(
        body,
        grid=(N // BL,),
        in_specs=[
            _node_spec(), _node_spec(), _node_spec(), _node_spec(),
            _node_spec(), _node_spec(),
            _full_spec((256, F)),
            _full_spec((1, F)),
            _full_spec((F, 2 * F)),
            _full_spec((1, 2 * F)),
        ],
        out_specs=pl.BlockSpec((ppb, 8, F), lambda i: (i, 0, 0)),
        out_shape=jax.ShapeDtypeStruct((PAIRS, 8, F), f32),
    )(s0, s1, b, h, kc, dg, wupd, bupd, wagg, bagg)


def kernel(node_features, edge_features, from_idx, to_idx, graph_idx,
           batch_data_sizes, W_enc_node, b_enc_node, W_enc_edge, b_enc_edge,
           W_msg, b_msg, W_upd, b_upd, W_agg, b_agg):
    f32 = jnp.float32
    i32 = jnp.int32
    ben = b_enc_node.reshape(1, F)
    bee = b_enc_edge.reshape(1, 16)
    bmsg = b_msg.reshape(1, F)
    bupd = b_upd.reshape(1, F)
    bagg = b_agg.reshape(1, 2 * F)

    npair_pad = NTILES * PPT - 2 * E
    src_all = jnp.concatenate(
        [from_idx, to_idx, jnp.zeros((npair_pad,), i32)]).reshape(IDXROWS, BW)
    dst_all = jnp.concatenate(
        [to_idx, from_idx, jnp.full((npair_pad,), DUMMY, i32)]).reshape(IDXROWS, BW)

    eids = jnp.arange(E, dtype=i32)
    esrc = jnp.concatenate(
        [eids, eids, jnp.zeros((npair_pad,), i32)]).reshape(IDXROWS, BW)
    ef128 = jnp.concatenate(
        [edge_features, jnp.ones((E, 1), f32), jnp.zeros((E, F - 17), f32)], axis=1)

    z_f = jnp.zeros((NPAD, F), f32)

    efa = _spmm_sc(ef128, esrc, dst_all, z_f)
    h, a, b, kc, dg = _tc_pre(node_features, efa[0], efa[1], W_enc_node, ben,
                              W_enc_edge, bee, W_msg, bmsg)
    for _ in range(2):
        s = _spmm_sc(a, src_all, dst_all, z_f)
        h, a, b = _tc_update(s[0], s[1], b, h, kc, dg, W_upd, bupd, W_msg)
    s = _spmm_sc(a, src_all, dst_all, z_f)
    out = _tc_update_final(s[0], s[1], b, h, kc, dg, W_upd, bupd, W_agg, bagg)
    return out[:, 0, 0]
